# baseline (device time: 103726 ns/iter reference)
import jax
import jax.numpy as jnp
from jax import lax
from jax.experimental import pallas as pl
from jax.experimental.pallas import tpu as pltpu

S = 1024
D = 2048
DC = 128
H = 16
HL = 8
DH = 128
DR = 32
DL = HL * DH

_VMEM = pl.BlockSpec(memory_space=pltpu.VMEM)


def _dot(a, b):
    return lax.dot_general(
        a, b, (((1,), (0,)), ((), ())), preferred_element_type=jnp.float32)


def _dot_nt(a, b):
    return lax.dot_general(
        a, b, (((1,), (1,)), ((), ())), preferred_element_type=jnp.float32)


def kernel(x, Wdkv, Wuk, Wuv, Wq, Wqr, Wkr, Wo):
    x2 = x.reshape(S, D)
    bf16 = jnp.bfloat16

    def body_a(x_ref, wdkv_ref, wuk_ref, wuv_ref, wqr_ref, wkr_ref, wq_ref,
               k_ref, v_ref, qr_ref, kr_ref, q_ref,
               c_ref, c_rx_ref, wuk_b_ref, wuk_rx_ref, wuv_b_ref, wuv_rx_ref,
               send_sems, recv_sems):
        my_x = lax.axis_index("x")
        my_y = lax.axis_index("y")
        my_z = lax.axis_index("z")
        nbr = (1 - my_x, my_y, my_z)

        barrier = pltpu.get_barrier_semaphore()
        pl.semaphore_signal(barrier, inc=1, device_id=nbr,
                            device_id_type=pl.DeviceIdType.MESH)
        pl.semaphore_wait(barrier, 1)

        wuk_b_ref[...] = wuk_ref[...].astype(bf16)
        wuv_b_ref[...] = wuv_ref[...].astype(bf16)

        def _rdma(i, src, dst):
            r = pltpu.make_async_remote_copy(
                src_ref=src, dst_ref=dst,
                send_sem=send_sems.at[i], recv_sem=recv_sems.at[i],
                device_id=nbr, device_id_type=pl.DeviceIdType.MESH)
            r.start()
            return r

        r_wuk = _rdma(0, wuk_b_ref, wuk_rx_ref)
        r_wuv = _rdma(1, wuv_b_ref, wuv_rx_ref)

        c_ref[...] = _dot(x_ref[...], wdkv_ref[...]).astype(bf16)
        r_c = _rdma(2, c_ref, c_rx_ref)

        kr_ref[...] = _dot(x_ref[...], wkr_ref[...]).astype(bf16)
        for side in (0, 1):
            @pl.when(my_x == side)
            def _(side=side):
                qr_ref[...] = _dot(
                    x_ref[...],
                    wqr_ref[:, side * HL * DR:(side + 1) * HL * DR],
                ).astype(bf16)
                q_ref[...] = _dot(
                    x_ref[...], wq_ref[:, side * DL:(side + 1) * DL],
                ).astype(bf16)

        r_wuk.wait()
        r_wuv.wait()
        r_c.wait()

        for side in (0, 1):
            @pl.when(my_x == side)
            def _(side=side):
                lo, hi = side * DL, (side + 1) * DL
                k_ref[...] = (_dot(c_ref[...], wuk_b_ref[:, lo:hi])
                              + _dot(c_rx_ref[...], wuk_rx_ref[:, lo:hi])
                              ).astype(bf16)
                v_ref[...] = (_dot(c_ref[...], wuv_b_ref[:, lo:hi])
                              + _dot(c_rx_ref[...], wuv_rx_ref[:, lo:hi])
                              ).astype(bf16)

    k, v, qr, kr, q = pl.pallas_call(
        body_a,
        out_shape=[
            jax.ShapeDtypeStruct((S, DL), bf16),
            jax.ShapeDtypeStruct((S, DL), bf16),
            jax.ShapeDtypeStruct((S, HL * DR), bf16),
            jax.ShapeDtypeStruct((S, DR), bf16),
            jax.ShapeDtypeStruct((S, DL), bf16),
        ],
        in_specs=[_VMEM] * 7,
        out_specs=[_VMEM] * 5,
        scratch_shapes=[
            pltpu.VMEM((S, DC), bf16),
            pltpu.VMEM((S, DC), bf16),
            pltpu.VMEM((DC, D), bf16),
            pltpu.VMEM((DC, D), bf16),
            pltpu.VMEM((DC, D), bf16),
            pltpu.VMEM((DC, D), bf16),
            pltpu.SemaphoreType.DMA((3,)),
            pltpu.SemaphoreType.DMA((3,)),
        ],
        compiler_params=pltpu.CompilerParams(
            collective_id=0, vmem_limit_bytes=62 * 1024 * 1024),
    )(x2, Wdkv, Wuk, Wuv, Wqr, Wkr, Wq)

    def body_c(q_ref, qr_ref, kr_ref, k_ref, v_ref, o_ref):
        scale = (DH + DR) ** -0.5
        kr_all = kr_ref[...]
        for h in range(HL):
            qh = q_ref[:, h * DH:(h + 1) * DH]
            kh = k_ref[:, h * DH:(h + 1) * DH]
            qrh = qr_ref[:, h * DR:(h + 1) * DR]
            s = (_dot_nt(qh, kh) + _dot_nt(qrh, kr_all)) * scale
            e = jnp.exp(s)
            recip = 1.0 / jnp.sum(e, axis=1, keepdims=True)
            vh = v_ref[:, h * DH:(h + 1) * DH].astype(jnp.float32)
            o_ref[:, h * DH:(h + 1) * DH] = (_dot(e, vh) * recip).astype(bf16)

    o = pl.pallas_call(
        body_c,
        out_shape=jax.ShapeDtypeStruct((S, DL), bf16),
        in_specs=[_VMEM] * 5,
        out_specs=_VMEM,
        compiler_params=pltpu.CompilerParams(
            vmem_limit_bytes=62 * 1024 * 1024),
    )(q, qr, kr, k, v)

    def body_d(o_ref, wo_ref, out_ref, o_rx_ref, send_sem, recv_sem):
        my_x = lax.axis_index("x")
        my_y = lax.axis_index("y")
        my_z = lax.axis_index("z")
        nbr = (1 - my_x, my_y, my_z)

        barrier = pltpu.get_barrier_semaphore()
        pl.semaphore_signal(barrier, inc=1, device_id=nbr,
                            device_id_type=pl.DeviceIdType.MESH)
        pl.semaphore_wait(barrier, 1)

        r = pltpu.make_async_remote_copy(
            src_ref=o_ref, dst_ref=o_rx_ref,
            send_sem=send_sem, recv_sem=recv_sem,
            device_id=nbr, device_id_type=pl.DeviceIdType.MESH)
        r.start()

        for side in (0, 1):
            @pl.when(my_x == side)
            def _(side=side):
                lo, hi = side * DL, (side + 1) * DL
                out_ref[...] = _dot(o_ref[...].astype(jnp.float32),
                                    wo_ref[lo:hi, :])

        r.wait()

        for side in (0, 1):
            @pl.when(my_x == side)
            def _(side=side):
                lo, hi = (1 - side) * DL, (2 - side) * DL
                out_ref[...] += _dot(o_rx_ref[...].astype(jnp.float32),
                                     wo_ref[lo:hi, :])

    out = pl.pallas_call(
        body_d,
        out_shape=jax.ShapeDtypeStruct((S, D), jnp.float32),
        in_specs=[_VMEM] * 2,
        out_specs=_VMEM,
        scratch_shapes=[
            pltpu.VMEM((S, DL), bf16),
            pltpu.SemaphoreType.DMA,
            pltpu.SemaphoreType.DMA,
        ],
        compiler_params=pltpu.CompilerParams(
            collective_id=1, vmem_limit_bytes=62 * 1024 * 1024),
    )(o, Wo)

    return out.reshape(1, S, D)


# device time: 86768 ns/iter; 1.1954x vs baseline; 1.1954x over previous
import jax
import jax.numpy as jnp
from jax import lax
from jax.experimental import pallas as pl
from jax.experimental.pallas import tpu as pltpu

S = 1024
D = 2048
DC = 128
H = 16
HL = 8
DH = 128
DR = 32
DL = HL * DH

_VMEM = pl.BlockSpec(memory_space=pltpu.VMEM)


def _dot(a, b):
    return lax.dot_general(
        a, b, (((1,), (0,)), ((), ())), preferred_element_type=jnp.float32)


def _dot_nt(a, b):
    return lax.dot_general(
        a, b, (((1,), (1,)), ((), ())), preferred_element_type=jnp.float32)


def kernel(x, Wdkv, Wuk, Wuv, Wq, Wqr, Wkr, Wo):
    x2 = x.reshape(S, D)
    bf16 = jnp.bfloat16

    def body_a(x_ref, wdkv_ref, wuk_ref, wuv_ref, wqr_ref, wkr_ref, wq_ref,
               k_ref, v_ref, qr_ref, kr_ref, q_ref,
               c_ref, c_rx_ref, wuk_b_ref, wuk_rx_ref, wuv_b_ref, wuv_rx_ref,
               send_sems, recv_sems):
        my_x = lax.axis_index("x")
        my_y = lax.axis_index("y")
        my_z = lax.axis_index("z")
        nbr = (1 - my_x, my_y, my_z)

        barrier = pltpu.get_barrier_semaphore()
        pl.semaphore_signal(barrier, inc=1, device_id=nbr,
                            device_id_type=pl.DeviceIdType.MESH)
        pl.semaphore_wait(barrier, 1)

        wuk_b_ref[...] = wuk_ref[...].astype(bf16)
        wuv_b_ref[...] = wuv_ref[...].astype(bf16)

        def _rdma(i, src, dst):
            r = pltpu.make_async_remote_copy(
                src_ref=src, dst_ref=dst,
                send_sem=send_sems.at[i], recv_sem=recv_sems.at[i],
                device_id=nbr, device_id_type=pl.DeviceIdType.MESH)
            r.start()
            return r

        r_wuk = _rdma(0, wuk_b_ref, wuk_rx_ref)
        r_wuv = _rdma(1, wuv_b_ref, wuv_rx_ref)

        c_ref[...] = _dot(x_ref[...], wdkv_ref[...]).astype(bf16)
        r_c = _rdma(2, c_ref, c_rx_ref)

        kr_ref[...] = _dot(x_ref[...], wkr_ref[...]).astype(bf16)
        for side in (0, 1):
            @pl.when(my_x == side)
            def _(side=side):
                qr_ref[...] = _dot(
                    x_ref[...],
                    wqr_ref[:, side * HL * DR:(side + 1) * HL * DR],
                ).astype(bf16)
                q_ref[...] = _dot(
                    x_ref[...], wq_ref[:, side * DL:(side + 1) * DL],
                ).astype(bf16)

        r_wuk.wait()
        r_wuv.wait()
        r_c.wait()

        for side in (0, 1):
            @pl.when(my_x == side)
            def _(side=side):
                lo, hi = side * DL, (side + 1) * DL
                k_ref[...] = (_dot(c_ref[...], wuk_b_ref[:, lo:hi])
                              + _dot(c_rx_ref[...], wuk_rx_ref[:, lo:hi])
                              ).astype(bf16)
                v_ref[...] = (_dot(c_ref[...], wuv_b_ref[:, lo:hi])
                              + _dot(c_rx_ref[...], wuv_rx_ref[:, lo:hi])
                              ).astype(bf16)

    k, v, qr, kr, q = pl.pallas_call(
        body_a,
        out_shape=[
            jax.ShapeDtypeStruct((S, DL), bf16),
            jax.ShapeDtypeStruct((S, DL), bf16),
            jax.ShapeDtypeStruct((S, HL * DR), bf16),
            jax.ShapeDtypeStruct((S, DR), bf16),
            jax.ShapeDtypeStruct((S, DL), bf16),
        ],
        in_specs=[_VMEM] * 7,
        out_specs=[_VMEM] * 5,
        scratch_shapes=[
            pltpu.VMEM((S, DC), bf16),
            pltpu.VMEM((S, DC), bf16),
            pltpu.VMEM((DC, D), bf16),
            pltpu.VMEM((DC, D), bf16),
            pltpu.VMEM((DC, D), bf16),
            pltpu.VMEM((DC, D), bf16),
            pltpu.SemaphoreType.DMA((3,)),
            pltpu.SemaphoreType.DMA((3,)),
        ],
        compiler_params=pltpu.CompilerParams(
            collective_id=0, vmem_limit_bytes=62 * 1024 * 1024),
    )(x2, Wdkv, Wuk, Wuv, Wqr, Wkr, Wq)

    def body_cd(q_ref, qr_ref, kr_ref, k_ref, v_ref, wo_ref, out_ref,
                o_ref, o_rx_ref, send_sems, recv_sems):
        my_x = lax.axis_index("x")
        my_y = lax.axis_index("y")
        my_z = lax.axis_index("z")
        nbr = (1 - my_x, my_y, my_z)

        barrier = pltpu.get_barrier_semaphore()
        pl.semaphore_signal(barrier, inc=1, device_id=nbr,
                            device_id_type=pl.DeviceIdType.MESH)
        pl.semaphore_wait(barrier, 1)

        scale = (DH + DR) ** -0.5
        kr_all = kr_ref[...]
        rdmas = []
        for h in range(HL):
            qh = q_ref[:, h * DH:(h + 1) * DH]
            kh = k_ref[:, h * DH:(h + 1) * DH]
            qrh = qr_ref[:, h * DR:(h + 1) * DR]
            s = (_dot_nt(qh, kh) + _dot_nt(qrh, kr_all)) * scale
            e = jnp.exp(s)
            recip = 1.0 / jnp.sum(e, axis=1, keepdims=True)
            vh = v_ref[:, h * DH:(h + 1) * DH].astype(jnp.float32)
            o_ref[:, h * DH:(h + 1) * DH] = (_dot(e, vh) * recip).astype(bf16)
            r = pltpu.make_async_remote_copy(
                src_ref=o_ref.at[:, pl.ds(h * DH, DH)],
                dst_ref=o_rx_ref.at[:, pl.ds(h * DH, DH)],
                send_sem=send_sems.at[h], recv_sem=recv_sems.at[h],
                device_id=nbr, device_id_type=pl.DeviceIdType.MESH)
            r.start()
            rdmas.append(r)

        for side in (0, 1):
            @pl.when(my_x == side)
            def _(side=side):
                lo, hi = side * DL, (side + 1) * DL
                out_ref[0] = _dot(o_ref[...].astype(jnp.float32),
                                  wo_ref[lo:hi, :])

        for r in rdmas:
            r.wait()

        for side in (0, 1):
            @pl.when(my_x == side)
            def _(side=side):
                lo, hi = (1 - side) * DL, (2 - side) * DL
                out_ref[0] += _dot(o_rx_ref[...].astype(jnp.float32),
                                   wo_ref[lo:hi, :])

    out = pl.pallas_call(
        body_cd,
        out_shape=jax.ShapeDtypeStruct((1, S, D), jnp.float32),
        in_specs=[_VMEM] * 6,
        out_specs=_VMEM,
        scratch_shapes=[
            pltpu.VMEM((S, DL), bf16),
            pltpu.VMEM((S, DL), bf16),
            pltpu.SemaphoreType.DMA((HL,)),
            pltpu.SemaphoreType.DMA((HL,)),
        ],
        compiler_params=pltpu.CompilerParams(
            collective_id=1, vmem_limit_bytes=62 * 1024 * 1024),
    )(q, qr, kr, k, v, Wo)

    return out


# device time: 80754 ns/iter; 1.2845x vs baseline; 1.0745x over previous
import jax
import jax.numpy as jnp
from jax import lax
from jax.experimental import pallas as pl
from jax.experimental.pallas import tpu as pltpu

S = 1024
D = 2048
DC = 128
H = 16
HL = 8
DH = 128
DR = 32
DL = HL * DH

_VMEM = pl.BlockSpec(memory_space=pltpu.VMEM)


def _dot(a, b):
    return lax.dot_general(
        a, b, (((1,), (0,)), ((), ())), preferred_element_type=jnp.float32)


def _dot_nt(a, b):
    return lax.dot_general(
        a, b, (((1,), (1,)), ((), ())), preferred_element_type=jnp.float32)


def kernel(x, Wdkv, Wuk, Wuv, Wq, Wqr, Wkr, Wo):
    x2 = x.reshape(S, D)
    bf16 = jnp.bfloat16

    def body_a(x_ref, wdkv_ref, wuk_ref, wuv_ref, wqr_ref, wkr_ref, wq_ref,
               k_ref, v_ref, qr_ref, kr_ref, q_ref,
               c_ref, c_rx_ref, wuk_b_ref, wuk_rx_ref, wuv_b_ref, wuv_rx_ref,
               wq_v_ref, wqr_v_ref, copy_sems, send_sems, recv_sems):
        my_x = lax.axis_index("x")
        my_y = lax.axis_index("y")
        my_z = lax.axis_index("z")
        nbr = (1 - my_x, my_y, my_z)

        barrier = pltpu.get_barrier_semaphore()
        pl.semaphore_signal(barrier, inc=1, device_id=nbr,
                            device_id_type=pl.DeviceIdType.MESH)
        pl.semaphore_wait(barrier, 1)

        wuk_b_ref[...] = wuk_ref[...].astype(bf16)
        wuv_b_ref[...] = wuv_ref[...].astype(bf16)

        def _rdma(i, src, dst):
            r = pltpu.make_async_remote_copy(
                src_ref=src, dst_ref=dst,
                send_sem=send_sems.at[i], recv_sem=recv_sems.at[i],
                device_id=nbr, device_id_type=pl.DeviceIdType.MESH)
            r.start()
            return r

        r_wuk = _rdma(0, wuk_b_ref, wuk_rx_ref)
        r_wuv = _rdma(1, wuv_b_ref, wuv_rx_ref)

        cp_wq = pltpu.make_async_copy(
            wq_ref.at[:, pl.ds(my_x * DL, DL)], wq_v_ref, copy_sems.at[0])
        cp_wq.start()
        cp_wqr = pltpu.make_async_copy(
            wqr_ref.at[:, pl.ds(my_x * HL * DR, HL * DR)], wqr_v_ref,
            copy_sems.at[1])
        cp_wqr.start()

        c_ref[...] = _dot(x_ref[...], wdkv_ref[...]).astype(bf16)
        r_c = _rdma(2, c_ref, c_rx_ref)

        kr_ref[...] = _dot(x_ref[...], wkr_ref[...]).astype(bf16)
        cp_wqr.wait()
        qr_ref[...] = _dot(x_ref[...], wqr_v_ref[...]).astype(bf16)
        cp_wq.wait()
        q_ref[...] = _dot(x_ref[...], wq_v_ref[...]).astype(bf16)

        r_wuk.wait()
        r_wuv.wait()
        r_c.wait()

        for side in (0, 1):
            @pl.when(my_x == side)
            def _(side=side):
                lo, hi = side * DL, (side + 1) * DL
                k_ref[...] = (_dot(c_ref[...], wuk_b_ref[:, lo:hi])
                              + _dot(c_rx_ref[...], wuk_rx_ref[:, lo:hi])
                              ).astype(bf16)
                v_ref[...] = (_dot(c_ref[...], wuv_b_ref[:, lo:hi])
                              + _dot(c_rx_ref[...], wuv_rx_ref[:, lo:hi])
                              ).astype(bf16)

    k, v, qr, kr, q = pl.pallas_call(
        body_a,
        out_shape=[
            jax.ShapeDtypeStruct((S, DL), bf16),
            jax.ShapeDtypeStruct((S, DL), bf16),
            jax.ShapeDtypeStruct((S, HL * DR), bf16),
            jax.ShapeDtypeStruct((S, DR), bf16),
            jax.ShapeDtypeStruct((S, DL), bf16),
        ],
        in_specs=[_VMEM] * 4
        + [pl.BlockSpec(memory_space=pltpu.MemorySpace.HBM), _VMEM,
           pl.BlockSpec(memory_space=pltpu.MemorySpace.HBM)],
        out_specs=[_VMEM] * 5,
        scratch_shapes=[
            pltpu.VMEM((S, DC), bf16),
            pltpu.VMEM((S, DC), bf16),
            pltpu.VMEM((DC, D), bf16),
            pltpu.VMEM((DC, D), bf16),
            pltpu.VMEM((DC, D), bf16),
            pltpu.VMEM((DC, D), bf16),
            pltpu.VMEM((D, DL), jnp.float32),
            pltpu.VMEM((D, HL * DR), jnp.float32),
            pltpu.SemaphoreType.DMA((2,)),
            pltpu.SemaphoreType.DMA((3,)),
            pltpu.SemaphoreType.DMA((3,)),
        ],
        compiler_params=pltpu.CompilerParams(
            collective_id=0, vmem_limit_bytes=62 * 1024 * 1024),
    )(x2, Wdkv, Wuk, Wuv, Wqr, Wkr, Wq)

    def body_cd(q_ref, qr_ref, kr_ref, k_ref, v_ref, wo_ref, out_ref,
                o_ref, o_rx_ref, send_sems, recv_sems):
        my_x = lax.axis_index("x")
        my_y = lax.axis_index("y")
        my_z = lax.axis_index("z")
        nbr = (1 - my_x, my_y, my_z)

        barrier = pltpu.get_barrier_semaphore()
        pl.semaphore_signal(barrier, inc=1, device_id=nbr,
                            device_id_type=pl.DeviceIdType.MESH)
        pl.semaphore_wait(barrier, 1)

        scale = (DH + DR) ** -0.5
        kr_all = kr_ref[...]
        rdmas = []
        for h in range(HL):
            qh = q_ref[:, h * DH:(h + 1) * DH]
            kh = k_ref[:, h * DH:(h + 1) * DH]
            qrh = qr_ref[:, h * DR:(h + 1) * DR]
            s = (_dot_nt(qh, kh) + _dot_nt(qrh, kr_all)) * scale
            e = jnp.exp(s)
            recip = 1.0 / jnp.sum(e, axis=1, keepdims=True)
            vh = v_ref[:, h * DH:(h + 1) * DH].astype(jnp.float32)
            o_ref[:, h * DH:(h + 1) * DH] = (_dot(e, vh) * recip).astype(bf16)
            r = pltpu.make_async_remote_copy(
                src_ref=o_ref.at[:, pl.ds(h * DH, DH)],
                dst_ref=o_rx_ref.at[:, pl.ds(h * DH, DH)],
                send_sem=send_sems.at[h], recv_sem=recv_sems.at[h],
                device_id=nbr, device_id_type=pl.DeviceIdType.MESH)
            r.start()
            rdmas.append(r)

        for side in (0, 1):
            @pl.when(my_x == side)
            def _(side=side):
                lo, hi = side * DL, (side + 1) * DL
                out_ref[0] = _dot(o_ref[...].astype(jnp.float32),
                                  wo_ref[lo:hi, :])

        for r in rdmas:
            r.wait()

        for side in (0, 1):
            @pl.when(my_x == side)
            def _(side=side):
                lo, hi = (1 - side) * DL, (2 - side) * DL
                out_ref[0] += _dot(o_rx_ref[...].astype(jnp.float32),
                                   wo_ref[lo:hi, :])

    out = pl.pallas_call(
        body_cd,
        out_shape=jax.ShapeDtypeStruct((1, S, D), jnp.float32),
        in_specs=[_VMEM] * 6,
        out_specs=_VMEM,
        scratch_shapes=[
            pltpu.VMEM((S, DL), bf16),
            pltpu.VMEM((S, DL), bf16),
            pltpu.SemaphoreType.DMA((HL,)),
            pltpu.SemaphoreType.DMA((HL,)),
        ],
        compiler_params=pltpu.CompilerParams(
            collective_id=1, vmem_limit_bytes=62 * 1024 * 1024),
    )(q, qr, kr, k, v, Wo)

    return out


# device time: 76027 ns/iter; 1.3643x vs baseline; 1.0622x over previous
import jax
import jax.numpy as jnp
from jax import lax
from jax.experimental import pallas as pl
from jax.experimental.pallas import tpu as pltpu

S = 1024
D = 2048
DC = 128
H = 16
HL = 8
DH = 128
DR = 32
DL = HL * DH

_VMEM = pl.BlockSpec(memory_space=pltpu.VMEM)


def _dot(a, b):
    return lax.dot_general(
        a, b, (((1,), (0,)), ((), ())), preferred_element_type=jnp.float32)


def _dot_nt(a, b):
    return lax.dot_general(
        a, b, (((1,), (1,)), ((), ())), preferred_element_type=jnp.float32)


def kernel(x, Wdkv, Wuk, Wuv, Wq, Wqr, Wkr, Wo):
    x2 = x.reshape(S, D)
    bf16 = jnp.bfloat16

    def body_a(x_ref, wdkv_ref, wuk_ref, wuv_ref, wqr_ref, wkr_ref, wq_ref,
               k_ref, v_ref, qr_ref, kr_ref, q_ref,
               c_ref, c_rx_ref, wuk_b_ref, wuk_rx_ref, wuv_b_ref, wuv_rx_ref,
               wq_v_ref, wqr_v_ref, copy_sems, send_sems, recv_sems):
        my_x = lax.axis_index("x")
        my_y = lax.axis_index("y")
        my_z = lax.axis_index("z")
        nbr = (1 - my_x, my_y, my_z)

        barrier = pltpu.get_barrier_semaphore()
        pl.semaphore_signal(barrier, inc=1, device_id=nbr,
                            device_id_type=pl.DeviceIdType.MESH)
        pl.semaphore_wait(barrier, 1)

        wuk_b_ref[...] = wuk_ref[...].astype(bf16)
        wuv_b_ref[...] = wuv_ref[...].astype(bf16)

        def _rdma(i, src, dst):
            r = pltpu.make_async_remote_copy(
                src_ref=src, dst_ref=dst,
                send_sem=send_sems.at[i], recv_sem=recv_sems.at[i],
                device_id=nbr, device_id_type=pl.DeviceIdType.MESH)
            r.start()
            return r

        r_wuk = _rdma(0, wuk_b_ref, wuk_rx_ref)
        r_wuv = _rdma(1, wuv_b_ref, wuv_rx_ref)

        cp_wq = pltpu.make_async_copy(
            wq_ref.at[:, pl.ds(my_x * DL, DL)], wq_v_ref, copy_sems.at[0])
        cp_wq.start()
        cp_wqr = pltpu.make_async_copy(
            wqr_ref.at[:, pl.ds(my_x * HL * DR, HL * DR)], wqr_v_ref,
            copy_sems.at[1])
        cp_wqr.start()

        c_ref[...] = _dot(x_ref[...], wdkv_ref[...]).astype(bf16)
        r_c = _rdma(2, c_ref, c_rx_ref)

        kr_ref[...] = _dot(x_ref[...], wkr_ref[...]).astype(bf16)
        cp_wqr.wait()
        qr_ref[...] = _dot(x_ref[...], wqr_v_ref[...]).astype(bf16)
        cp_wq.wait()
        q_ref[...] = _dot(x_ref[...], wq_v_ref[...]).astype(bf16)

        r_wuk.wait()
        r_wuv.wait()
        r_c.wait()

        for side in (0, 1):
            @pl.when(my_x == side)
            def _(side=side):
                lo, hi = side * DL, (side + 1) * DL
                k_ref[...] = (_dot(c_ref[...], wuk_b_ref[:, lo:hi])
                              + _dot(c_rx_ref[...], wuk_rx_ref[:, lo:hi])
                              ).astype(bf16)
                v_ref[...] = (_dot(c_ref[...], wuv_b_ref[:, lo:hi])
                              + _dot(c_rx_ref[...], wuv_rx_ref[:, lo:hi])
                              ).astype(bf16)

    k, v, qr, kr, q = pl.pallas_call(
        body_a,
        out_shape=[
            jax.ShapeDtypeStruct((S, DL), bf16),
            jax.ShapeDtypeStruct((S, DL), bf16),
            jax.ShapeDtypeStruct((S, HL * DR), bf16),
            jax.ShapeDtypeStruct((S, DR), bf16),
            jax.ShapeDtypeStruct((S, DL), bf16),
        ],
        in_specs=[_VMEM] * 4
        + [pl.BlockSpec(memory_space=pltpu.MemorySpace.HBM), _VMEM,
           pl.BlockSpec(memory_space=pltpu.MemorySpace.HBM)],
        out_specs=[_VMEM] * 5,
        scratch_shapes=[
            pltpu.VMEM((S, DC), bf16),
            pltpu.VMEM((S, DC), bf16),
            pltpu.VMEM((DC, D), bf16),
            pltpu.VMEM((DC, D), bf16),
            pltpu.VMEM((DC, D), bf16),
            pltpu.VMEM((DC, D), bf16),
            pltpu.VMEM((D, DL), jnp.float32),
            pltpu.VMEM((D, HL * DR), jnp.float32),
            pltpu.SemaphoreType.DMA((2,)),
            pltpu.SemaphoreType.DMA((3,)),
            pltpu.SemaphoreType.DMA((3,)),
        ],
        compiler_params=pltpu.CompilerParams(
            collective_id=0, vmem_limit_bytes=62 * 1024 * 1024),
    )(x2, Wdkv, Wuk, Wuv, Wqr, Wkr, Wq)

    def body_cd(q_ref, qr_ref, kr_ref, k_ref, v_ref, wo_ref, out_ref,
                o_ref, o_rx_ref, wo_v_ref, copy_sem, send_sems, recv_sems):
        my_x = lax.axis_index("x")
        my_y = lax.axis_index("y")
        my_z = lax.axis_index("z")
        nbr = (1 - my_x, my_y, my_z)

        barrier = pltpu.get_barrier_semaphore()
        pl.semaphore_signal(barrier, inc=1, device_id=nbr,
                            device_id_type=pl.DeviceIdType.MESH)
        pl.semaphore_wait(barrier, 1)

        cp_wo = pltpu.make_async_copy(wo_ref, wo_v_ref, copy_sem)
        cp_wo.start()

        scale = (DH + DR) ** -0.5
        kr_all = kr_ref[...]
        rdmas = []
        for h in range(HL):
            qh = q_ref[:, h * DH:(h + 1) * DH]
            kh = k_ref[:, h * DH:(h + 1) * DH]
            qrh = qr_ref[:, h * DR:(h + 1) * DR]
            s = (_dot_nt(qh, kh) + _dot_nt(qrh, kr_all)) * scale
            e = jnp.exp(s)
            recip = 1.0 / jnp.sum(e, axis=1, keepdims=True)
            vh = v_ref[:, h * DH:(h + 1) * DH].astype(jnp.float32)
            o_ref[:, h * DH:(h + 1) * DH] = (_dot(e, vh) * recip).astype(bf16)
            r = pltpu.make_async_remote_copy(
                src_ref=o_ref.at[:, pl.ds(h * DH, DH)],
                dst_ref=o_rx_ref.at[:, pl.ds(h * DH, DH)],
                send_sem=send_sems.at[h], recv_sem=recv_sems.at[h],
                device_id=nbr, device_id_type=pl.DeviceIdType.MESH)
            r.start()
            rdmas.append(r)

        cp_wo.wait()
        for side in (0, 1):
            @pl.when(my_x == side)
            def _(side=side):
                lo, hi = side * DL, (side + 1) * DL
                out_ref[0] = _dot(o_ref[...].astype(jnp.float32),
                                  wo_v_ref[lo:hi, :])

        for r in rdmas:
            r.wait()

        for side in (0, 1):
            @pl.when(my_x == side)
            def _(side=side):
                lo, hi = (1 - side) * DL, (2 - side) * DL
                out_ref[0] += _dot(o_rx_ref[...].astype(jnp.float32),
                                   wo_v_ref[lo:hi, :])

    out = pl.pallas_call(
        body_cd,
        out_shape=jax.ShapeDtypeStruct((1, S, D), jnp.float32),
        in_specs=[_VMEM] * 5
        + [pl.BlockSpec(memory_space=pltpu.MemorySpace.HBM)],
        out_specs=_VMEM,
        scratch_shapes=[
            pltpu.VMEM((S, DL), bf16),
            pltpu.VMEM((S, DL), bf16),
            pltpu.VMEM((D, D), jnp.float32),
            pltpu.SemaphoreType.DMA,
            pltpu.SemaphoreType.DMA((HL,)),
            pltpu.SemaphoreType.DMA((HL,)),
        ],
        compiler_params=pltpu.CompilerParams(
            collective_id=1, vmem_limit_bytes=62 * 1024 * 1024),
    )(q, qr, kr, k, v, Wo)

    return out


# device time: 73495 ns/iter; 1.4113x vs baseline; 1.0345x over previous
import jax
import jax.numpy as jnp
from jax import lax
from jax.experimental import pallas as pl
from jax.experimental.pallas import tpu as pltpu

S = 1024
D = 2048
DC = 128
H = 16
HL = 8
DH = 128
DR = 32
DL = HL * DH

_VMEM = pl.BlockSpec(memory_space=pltpu.VMEM)


def _dot(a, b):
    return lax.dot_general(
        a, b, (((1,), (0,)), ((), ())), preferred_element_type=jnp.float32)


def _dot_nt(a, b):
    return lax.dot_general(
        a, b, (((1,), (1,)), ((), ())), preferred_element_type=jnp.float32)


def kernel(x, Wdkv, Wuk, Wuv, Wq, Wqr, Wkr, Wo):
    x2 = x.reshape(S, D)
    bf16 = jnp.bfloat16

    def body_a(x_ref, wdkv_ref, wuk_ref, wuv_ref, wqr_ref, wkr_ref, wq_ref,
               k_ref, v_ref, qr_ref, kr_ref, q_ref,
               c_ref, c_rx_ref, wuk_b_ref, wuk_rx_ref, wuv_b_ref, wuv_rx_ref,
               wq_v_ref, wqr_v_ref, x_v_ref, copy_sems, send_sems, recv_sems):
        my_x = lax.axis_index("x")
        my_y = lax.axis_index("y")
        my_z = lax.axis_index("z")
        nbr = (1 - my_x, my_y, my_z)

        scale = (DH + DR) ** -0.5

        cp_x0 = pltpu.make_async_copy(
            x_ref.at[pl.ds(0, S // 2)], x_v_ref.at[pl.ds(0, S // 2)],
            copy_sems.at[2])
        cp_x0.start()
        cp_x1 = pltpu.make_async_copy(
            x_ref.at[pl.ds(S // 2, S // 2)], x_v_ref.at[pl.ds(S // 2, S // 2)],
            copy_sems.at[3])
        cp_x1.start()

        barrier = pltpu.get_barrier_semaphore()
        pl.semaphore_signal(barrier, inc=1, device_id=nbr,
                            device_id_type=pl.DeviceIdType.MESH)
        pl.semaphore_wait(barrier, 1)

        wuk_b_ref[...] = wuk_ref[...].astype(bf16)
        wuv_b_ref[...] = wuv_ref[...].astype(bf16)

        def _rdma(i, src, dst):
            r = pltpu.make_async_remote_copy(
                src_ref=src, dst_ref=dst,
                send_sem=send_sems.at[i], recv_sem=recv_sems.at[i],
                device_id=nbr, device_id_type=pl.DeviceIdType.MESH)
            r.start()
            return r

        r_wuk = _rdma(0, wuk_b_ref, wuk_rx_ref)
        r_wuv = _rdma(1, wuv_b_ref, wuv_rx_ref)

        cp_wq = pltpu.make_async_copy(
            wq_ref.at[:, pl.ds(my_x * DL, DL)], wq_v_ref, copy_sems.at[0])
        cp_wq.start()
        cp_wqr = pltpu.make_async_copy(
            wqr_ref.at[:, pl.ds(my_x * HL * DR, HL * DR)], wqr_v_ref,
            copy_sems.at[1])
        cp_wqr.start()

        cp_x0.wait()
        c_ref[0:S // 2] = _dot(x_v_ref[0:S // 2], wdkv_ref[...]).astype(bf16)
        cp_x1.wait()
        c_ref[S // 2:] = _dot(x_v_ref[S // 2:], wdkv_ref[...]).astype(bf16)
        r_c = _rdma(2, c_ref, c_rx_ref)

        kr_ref[...] = _dot(x_v_ref[...], wkr_ref[...]).astype(bf16)
        cp_wqr.wait()
        qr_ref[...] = (_dot(x_v_ref[...], wqr_v_ref[...]) * scale).astype(bf16)
        cp_wq.wait()
        q_ref[...] = (_dot(x_v_ref[...], wq_v_ref[...]) * scale).astype(bf16)

        r_wuk.wait()
        r_wuv.wait()
        r_c.wait()

        for side in (0, 1):
            @pl.when(my_x == side)
            def _(side=side):
                lo, hi = side * DL, (side + 1) * DL
                k_ref[...] = (_dot(c_ref[...], wuk_b_ref[:, lo:hi])
                              + _dot(c_rx_ref[...], wuk_rx_ref[:, lo:hi])
                              ).astype(bf16)
                v_ref[...] = (_dot(c_ref[...], wuv_b_ref[:, lo:hi])
                              + _dot(c_rx_ref[...], wuv_rx_ref[:, lo:hi])
                              ).astype(bf16)

    k, v, qr, kr, q = pl.pallas_call(
        body_a,
        out_shape=[
            jax.ShapeDtypeStruct((S, DL), bf16),
            jax.ShapeDtypeStruct((S, DL), bf16),
            jax.ShapeDtypeStruct((S, HL * DR), bf16),
            jax.ShapeDtypeStruct((S, DR), bf16),
            jax.ShapeDtypeStruct((S, DL), bf16),
        ],
        in_specs=[pl.BlockSpec(memory_space=pltpu.MemorySpace.HBM)]
        + [_VMEM] * 3
        + [pl.BlockSpec(memory_space=pltpu.MemorySpace.HBM), _VMEM,
           pl.BlockSpec(memory_space=pltpu.MemorySpace.HBM)],
        out_specs=[_VMEM] * 5,
        scratch_shapes=[
            pltpu.VMEM((S, DC), bf16),
            pltpu.VMEM((S, DC), bf16),
            pltpu.VMEM((DC, D), bf16),
            pltpu.VMEM((DC, D), bf16),
            pltpu.VMEM((DC, D), bf16),
            pltpu.VMEM((DC, D), bf16),
            pltpu.VMEM((D, DL), jnp.float32),
            pltpu.VMEM((D, HL * DR), jnp.float32),
            pltpu.VMEM((S, D), jnp.float32),
            pltpu.SemaphoreType.DMA((4,)),
            pltpu.SemaphoreType.DMA((3,)),
            pltpu.SemaphoreType.DMA((3,)),
        ],
        compiler_params=pltpu.CompilerParams(
            collective_id=0, vmem_limit_bytes=62 * 1024 * 1024),
    )(x2, Wdkv, Wuk, Wuv, Wqr, Wkr, Wq)

    def body_cd(q_ref, qr_ref, kr_ref, k_ref, v_ref, wo_ref, out_ref,
                o_ref, o_rx_ref, wo_v_ref, copy_sem, send_sems, recv_sems):
        my_x = lax.axis_index("x")
        my_y = lax.axis_index("y")
        my_z = lax.axis_index("z")
        nbr = (1 - my_x, my_y, my_z)

        barrier = pltpu.get_barrier_semaphore()
        pl.semaphore_signal(barrier, inc=1, device_id=nbr,
                            device_id_type=pl.DeviceIdType.MESH)
        pl.semaphore_wait(barrier, 1)

        cp_wo = pltpu.make_async_copy(wo_ref, wo_v_ref, copy_sem)
        cp_wo.start()

        kr_all = kr_ref[...]
        rdmas = []
        for h in range(HL):
            qh = q_ref[:, h * DH:(h + 1) * DH]
            kh = k_ref[:, h * DH:(h + 1) * DH]
            qrh = qr_ref[:, h * DR:(h + 1) * DR]
            s = _dot_nt(qh, kh) + _dot_nt(qrh, kr_all)
            e = jnp.exp(s)
            recip = 1.0 / jnp.sum(e, axis=1, keepdims=True)
            vh = v_ref[:, h * DH:(h + 1) * DH].astype(jnp.float32)
            o_ref[:, h * DH:(h + 1) * DH] = (_dot(e, vh) * recip).astype(bf16)
            r = pltpu.make_async_remote_copy(
                src_ref=o_ref.at[:, pl.ds(h * DH, DH)],
                dst_ref=o_rx_ref.at[:, pl.ds(h * DH, DH)],
                send_sem=send_sems.at[h], recv_sem=recv_sems.at[h],
                device_id=nbr, device_id_type=pl.DeviceIdType.MESH)
            r.start()
            rdmas.append(r)

        cp_wo.wait()
        for side in (0, 1):
            @pl.when(my_x == side)
            def _(side=side):
                lo, hi = side * DL, (side + 1) * DL
                out_ref[0] = _dot(o_ref[...].astype(jnp.float32),
                                  wo_v_ref[lo:hi, :])

        for r in rdmas:
            r.wait()

        for side in (0, 1):
            @pl.when(my_x == side)
            def _(side=side):
                lo, hi = (1 - side) * DL, (2 - side) * DL
                out_ref[0] += _dot(o_rx_ref[...].astype(jnp.float32),
                                   wo_v_ref[lo:hi, :])

    out = pl.pallas_call(
        body_cd,
        out_shape=jax.ShapeDtypeStruct((1, S, D), jnp.float32),
        in_specs=[_VMEM] * 5
        + [pl.BlockSpec(memory_space=pltpu.MemorySpace.HBM)],
        out_specs=_VMEM,
        scratch_shapes=[
            pltpu.VMEM((S, DL), bf16),
            pltpu.VMEM((S, DL), bf16),
            pltpu.VMEM((D, D), jnp.float32),
            pltpu.SemaphoreType.DMA,
            pltpu.SemaphoreType.DMA((HL,)),
            pltpu.SemaphoreType.DMA((HL,)),
        ],
        compiler_params=pltpu.CompilerParams(
            collective_id=1, vmem_limit_bytes=62 * 1024 * 1024),
    )(q, qr, kr, k, v, Wo)

    return out


# device time: 70408 ns/iter; 1.4732x vs baseline; 1.0438x over previous
import jax
import jax.numpy as jnp
from jax import lax
from jax.experimental import pallas as pl
from jax.experimental.pallas import tpu as pltpu

S = 1024
D = 2048
DC = 128
H = 16
HL = 8
DH = 128
DR = 32
DL = HL * DH

_VMEM = pl.BlockSpec(memory_space=pltpu.VMEM)


def _dot(a, b):
    return lax.dot_general(
        a, b, (((1,), (0,)), ((), ())), preferred_element_type=jnp.float32)


def _dot_nt(a, b):
    return lax.dot_general(
        a, b, (((1,), (1,)), ((), ())), preferred_element_type=jnp.float32)


def kernel(x, Wdkv, Wuk, Wuv, Wq, Wqr, Wkr, Wo):
    x2 = x.reshape(S, D)
    bf16 = jnp.bfloat16

    def body_a(x_ref, wdkv_ref, wuk_ref, wuv_ref, wqr_ref, wkr_ref, wq_ref,
               k_ref, v_ref, qr_ref, kr_ref, q_ref,
               c_ref, c_rx_ref, wuk_b_ref, wuk_rx_ref, wuv_b_ref, wuv_rx_ref,
               wq_v_ref, wqr_v_ref, x_v_ref, copy_sems, send_sems, recv_sems):
        my_x = lax.axis_index("x")
        my_y = lax.axis_index("y")
        my_z = lax.axis_index("z")
        nbr = (1 - my_x, my_y, my_z)

        scale = (DH + DR) ** -0.5

        cp_x0 = pltpu.make_async_copy(
            x_ref.at[pl.ds(0, S // 2)], x_v_ref.at[pl.ds(0, S // 2)],
            copy_sems.at[2])
        cp_x0.start()
        cp_x1 = pltpu.make_async_copy(
            x_ref.at[pl.ds(S // 2, S // 2)], x_v_ref.at[pl.ds(S // 2, S // 2)],
            copy_sems.at[3])
        cp_x1.start()

        barrier = pltpu.get_barrier_semaphore()
        pl.semaphore_signal(barrier, inc=1, device_id=nbr,
                            device_id_type=pl.DeviceIdType.MESH)
        pl.semaphore_wait(barrier, 1)

        wuk_b_ref[...] = wuk_ref[...].astype(bf16)
        wuv_b_ref[...] = wuv_ref[...].astype(bf16)

        def _rdma(i, src, dst):
            r = pltpu.make_async_remote_copy(
                src_ref=src, dst_ref=dst,
                send_sem=send_sems.at[i], recv_sem=recv_sems.at[i],
                device_id=nbr, device_id_type=pl.DeviceIdType.MESH)
            r.start()
            return r

        r_wuk = _rdma(0, wuk_b_ref, wuk_rx_ref)
        r_wuv = _rdma(1, wuv_b_ref, wuv_rx_ref)

        cp_wq = pltpu.make_async_copy(
            wq_ref.at[:, pl.ds(my_x * DL, DL)], wq_v_ref, copy_sems.at[0])
        cp_wq.start()
        cp_wqr = pltpu.make_async_copy(
            wqr_ref.at[:, pl.ds(my_x * HL * DR, HL * DR)], wqr_v_ref,
            copy_sems.at[1])
        cp_wqr.start()

        cp_x0.wait()
        c_ref[0:S // 2] = _dot(x_v_ref[0:S // 2], wdkv_ref[...]).astype(bf16)
        cp_x1.wait()
        c_ref[S // 2:] = _dot(x_v_ref[S // 2:], wdkv_ref[...]).astype(bf16)
        r_c = _rdma(2, c_ref, c_rx_ref)

        kr_ref[...] = _dot(x_v_ref[...], wkr_ref[...]).astype(bf16)
        cp_wqr.wait()
        qr_ref[...] = (_dot(x_v_ref[...], wqr_v_ref[...]) * scale).astype(bf16)
        cp_wq.wait()
        q_ref[...] = (_dot(x_v_ref[...], wq_v_ref[...]) * scale).astype(bf16)

        r_wuk.wait()
        r_wuv.wait()
        r_c.wait()

        for side in (0, 1):
            @pl.when(my_x == side)
            def _(side=side):
                lo, hi = side * DL, (side + 1) * DL
                k_ref[...] = (_dot(c_ref[...], wuk_b_ref[:, lo:hi])
                              + _dot(c_rx_ref[...], wuk_rx_ref[:, lo:hi])
                              ).astype(bf16)
                v_ref[...] = (_dot(c_ref[...], wuv_b_ref[:, lo:hi])
                              + _dot(c_rx_ref[...], wuv_rx_ref[:, lo:hi])
                              ).astype(bf16)

    k, v, qr, kr, q = pl.pallas_call(
        body_a,
        out_shape=[
            jax.ShapeDtypeStruct((S, DL), bf16),
            jax.ShapeDtypeStruct((S, DL), bf16),
            jax.ShapeDtypeStruct((S, HL * DR), bf16),
            jax.ShapeDtypeStruct((S, DR), bf16),
            jax.ShapeDtypeStruct((S, DL), bf16),
        ],
        in_specs=[pl.BlockSpec(memory_space=pltpu.MemorySpace.HBM)]
        + [_VMEM] * 3
        + [pl.BlockSpec(memory_space=pltpu.MemorySpace.HBM), _VMEM,
           pl.BlockSpec(memory_space=pltpu.MemorySpace.HBM)],
        out_specs=[_VMEM] * 5,
        scratch_shapes=[
            pltpu.VMEM((S, DC), bf16),
            pltpu.VMEM((S, DC), bf16),
            pltpu.VMEM((DC, D), bf16),
            pltpu.VMEM((DC, D), bf16),
            pltpu.VMEM((DC, D), bf16),
            pltpu.VMEM((DC, D), bf16),
            pltpu.VMEM((D, DL), jnp.float32),
            pltpu.VMEM((D, HL * DR), jnp.float32),
            pltpu.VMEM((S, D), jnp.float32),
            pltpu.SemaphoreType.DMA((4,)),
            pltpu.SemaphoreType.DMA((3,)),
            pltpu.SemaphoreType.DMA((3,)),
        ],
        compiler_params=pltpu.CompilerParams(
            collective_id=0, vmem_limit_bytes=62 * 1024 * 1024),
    )(x2, Wdkv, Wuk, Wuv, Wqr, Wkr, Wq)

    def body_cd(q_ref, qr_ref, kr_ref, k_ref, v_ref, wo_ref, out_ref,
                o_ref, o_rx_ref, wo_v_ref, acc_ref, copy_sem,
                send_sems, recv_sems):
        my_x = lax.axis_index("x")
        my_y = lax.axis_index("y")
        my_z = lax.axis_index("z")
        nbr = (1 - my_x, my_y, my_z)

        barrier = pltpu.get_barrier_semaphore()
        pl.semaphore_signal(barrier, inc=1, device_id=nbr,
                            device_id_type=pl.DeviceIdType.MESH)
        pl.semaphore_wait(barrier, 1)

        cp_wo = pltpu.make_async_copy(wo_ref, wo_v_ref, copy_sem)
        cp_wo.start()

        kr_all = kr_ref[...]
        rdmas = []
        for h in range(HL):
            qh = q_ref[:, h * DH:(h + 1) * DH]
            kh = k_ref[:, h * DH:(h + 1) * DH]
            qrh = qr_ref[:, h * DR:(h + 1) * DR]
            s = _dot_nt(qh, kh) + _dot_nt(qrh, kr_all)
            e = jnp.exp(s)
            recip = 1.0 / jnp.sum(e, axis=1, keepdims=True)
            vh = v_ref[:, h * DH:(h + 1) * DH].astype(jnp.float32)
            o_ref[:, h * DH:(h + 1) * DH] = (_dot(e, vh) * recip).astype(bf16)
            r = pltpu.make_async_remote_copy(
                src_ref=o_ref.at[:, pl.ds(h * DH, DH)],
                dst_ref=o_rx_ref.at[:, pl.ds(h * DH, DH)],
                send_sem=send_sems.at[h], recv_sem=recv_sems.at[h],
                device_id=nbr, device_id_type=pl.DeviceIdType.MESH)
            r.start()
            rdmas.append(r)

        cp_wo.wait()
        for side in (0, 1):
            @pl.when(my_x == side)
            def _(side=side):
                lo, hi = side * DL, (side + 1) * DL
                acc_ref[...] = _dot(o_ref[...].astype(jnp.float32),
                                    wo_v_ref[lo:hi, :])

        for r in rdmas:
            r.wait()

        for side in (0, 1):
            @pl.when(my_x == side)
            def _(side=side):
                lo, hi = (1 - side) * DL, (2 - side) * DL
                out_ref[0] = (acc_ref[...]
                              + _dot(o_rx_ref[...].astype(jnp.float32),
                                     wo_v_ref[lo:hi, :])).astype(bf16)

    out = pl.pallas_call(
        body_cd,
        out_shape=jax.ShapeDtypeStruct((1, S, D), bf16),
        in_specs=[_VMEM] * 5
        + [pl.BlockSpec(memory_space=pltpu.MemorySpace.HBM)],
        out_specs=_VMEM,
        scratch_shapes=[
            pltpu.VMEM((S, DL), bf16),
            pltpu.VMEM((S, DL), bf16),
            pltpu.VMEM((D, D), jnp.float32),
            pltpu.VMEM((S, D), jnp.float32),
            pltpu.SemaphoreType.DMA,
            pltpu.SemaphoreType.DMA((HL,)),
            pltpu.SemaphoreType.DMA((HL,)),
        ],
        compiler_params=pltpu.CompilerParams(
            collective_id=1, vmem_limit_bytes=62 * 1024 * 1024),
    )(q, qr, kr, k, v, Wo)

    return out


# device time: 69903 ns/iter; 1.4839x vs baseline; 1.0072x over previous
import jax
import jax.numpy as jnp
from jax import lax
from jax.experimental import pallas as pl
from jax.experimental.pallas import tpu as pltpu

S = 1024
D = 2048
DC = 128
H = 16
HL = 8
DH = 128
DR = 32
DL = HL * DH

_VMEM = pl.BlockSpec(memory_space=pltpu.VMEM)


def _dot(a, b):
    return lax.dot_general(
        a, b, (((1,), (0,)), ((), ())), preferred_element_type=jnp.float32)


def _dot_nt(a, b):
    return lax.dot_general(
        a, b, (((1,), (1,)), ((), ())), preferred_element_type=jnp.float32)


def kernel(x, Wdkv, Wuk, Wuv, Wq, Wqr, Wkr, Wo):
    x2 = x.reshape(S, D)
    bf16 = jnp.bfloat16

    def body_a(x_ref, wdkv_ref, wuk_ref, wuv_ref, wqr_ref, wkr_ref, wq_ref,
               k_ref, v_ref, qr_ref, kr_ref, q_ref,
               c_ref, c_rx_ref, wuk_b_ref, wuk_rx_ref, wuv_b_ref, wuv_rx_ref,
               wq_v_ref, wqr_v_ref, x_v_ref, wkr_v_ref, copy_sems,
               send_sems, recv_sems):
        my_x = lax.axis_index("x")
        my_y = lax.axis_index("y")
        my_z = lax.axis_index("z")
        nbr = (1 - my_x, my_y, my_z)

        scale = (DH + DR) ** -0.5

        cp_x0 = pltpu.make_async_copy(
            x_ref.at[pl.ds(0, S // 2)], x_v_ref.at[pl.ds(0, S // 2)],
            copy_sems.at[2])
        cp_x0.start()
        cp_x1 = pltpu.make_async_copy(
            x_ref.at[pl.ds(S // 2, S // 2)], x_v_ref.at[pl.ds(S // 2, S // 2)],
            copy_sems.at[3])
        cp_x1.start()

        barrier = pltpu.get_barrier_semaphore()
        pl.semaphore_signal(barrier, inc=1, device_id=nbr,
                            device_id_type=pl.DeviceIdType.MESH)
        pl.semaphore_wait(barrier, 1)

        wuk_b_ref[...] = wuk_ref[...].astype(bf16)
        wuv_b_ref[...] = wuv_ref[...].astype(bf16)

        def _rdma(i, src, dst):
            r = pltpu.make_async_remote_copy(
                src_ref=src, dst_ref=dst,
                send_sem=send_sems.at[i], recv_sem=recv_sems.at[i],
                device_id=nbr, device_id_type=pl.DeviceIdType.MESH)
            r.start()
            return r

        r_wuk = _rdma(0, wuk_b_ref, wuk_rx_ref)
        r_wuv = _rdma(1, wuv_b_ref, wuv_rx_ref)

        cp_wq = pltpu.make_async_copy(
            wq_ref.at[:, pl.ds(my_x * DL, DL)], wq_v_ref, copy_sems.at[0])
        cp_wq.start()
        cp_wqr = pltpu.make_async_copy(
            wqr_ref.at[:, pl.ds(my_x * HL * DR, HL * DR)], wqr_v_ref,
            copy_sems.at[1])
        cp_wqr.start()
        cp_wkr = pltpu.make_async_copy(wkr_ref, wkr_v_ref, copy_sems.at[4])
        cp_wkr.start()

        cp_x0.wait()
        c_ref[0:S // 2] = _dot(x_v_ref[0:S // 2], wdkv_ref[...]).astype(bf16)
        cp_x1.wait()
        c_ref[S // 2:] = _dot(x_v_ref[S // 2:], wdkv_ref[...]).astype(bf16)
        r_c = _rdma(2, c_ref, c_rx_ref)

        cp_wkr.wait()
        kr_ref[...] = _dot(x_v_ref[...], wkr_v_ref[...]).astype(bf16)
        cp_wqr.wait()
        qr_ref[...] = (_dot(x_v_ref[...], wqr_v_ref[...]) * scale).astype(bf16)
        cp_wq.wait()
        q_ref[...] = (_dot(x_v_ref[...], wq_v_ref[...]) * scale).astype(bf16)

        r_wuk.wait()
        r_wuv.wait()
        r_c.wait()

        for side in (0, 1):
            @pl.when(my_x == side)
            def _(side=side):
                lo, hi = side * DL, (side + 1) * DL
                k_ref[...] = (_dot(c_ref[...], wuk_b_ref[:, lo:hi])
                              + _dot(c_rx_ref[...], wuk_rx_ref[:, lo:hi])
                              ).astype(bf16)
                v_ref[...] = (_dot(c_ref[...], wuv_b_ref[:, lo:hi])
                              + _dot(c_rx_ref[...], wuv_rx_ref[:, lo:hi])
                              ).astype(bf16)

    k, v, qr, kr, q = pl.pallas_call(
        body_a,
        out_shape=[
            jax.ShapeDtypeStruct((S, DL), bf16),
            jax.ShapeDtypeStruct((S, DL), bf16),
            jax.ShapeDtypeStruct((S, HL * DR), bf16),
            jax.ShapeDtypeStruct((S, DR), bf16),
            jax.ShapeDtypeStruct((S, DL), bf16),
        ],
        in_specs=[pl.BlockSpec(memory_space=pltpu.MemorySpace.HBM)]
        + [_VMEM] * 3
        + [pl.BlockSpec(memory_space=pltpu.MemorySpace.HBM),
           pl.BlockSpec(memory_space=pltpu.MemorySpace.HBM),
           pl.BlockSpec(memory_space=pltpu.MemorySpace.HBM)],
        out_specs=[_VMEM] * 5,
        scratch_shapes=[
            pltpu.VMEM((S, DC), bf16),
            pltpu.VMEM((S, DC), bf16),
            pltpu.VMEM((DC, D), bf16),
            pltpu.VMEM((DC, D), bf16),
            pltpu.VMEM((DC, D), bf16),
            pltpu.VMEM((DC, D), bf16),
            pltpu.VMEM((D, DL), jnp.float32),
            pltpu.VMEM((D, HL * DR), jnp.float32),
            pltpu.VMEM((S, D), jnp.float32),
            pltpu.VMEM((D, DR), jnp.float32),
            pltpu.SemaphoreType.DMA((5,)),
            pltpu.SemaphoreType.DMA((3,)),
            pltpu.SemaphoreType.DMA((3,)),
        ],
        compiler_params=pltpu.CompilerParams(
            collective_id=0, vmem_limit_bytes=62 * 1024 * 1024),
    )(x2, Wdkv, Wuk, Wuv, Wqr, Wkr, Wq)

    def body_cd(q_ref, qr_ref, kr_ref, k_ref, v_ref, wo_ref, out_ref,
                o_ref, o_rx_ref, wo_v_ref, acc_ref, copy_sem,
                send_sems, recv_sems):
        my_x = lax.axis_index("x")
        my_y = lax.axis_index("y")
        my_z = lax.axis_index("z")
        nbr = (1 - my_x, my_y, my_z)

        barrier = pltpu.get_barrier_semaphore()
        pl.semaphore_signal(barrier, inc=1, device_id=nbr,
                            device_id_type=pl.DeviceIdType.MESH)
        pl.semaphore_wait(barrier, 1)

        cp_wo = pltpu.make_async_copy(wo_ref, wo_v_ref, copy_sem)
        cp_wo.start()

        kr_all = kr_ref[...]
        rdmas = []
        for h in range(HL):
            qh = q_ref[:, h * DH:(h + 1) * DH]
            kh = k_ref[:, h * DH:(h + 1) * DH]
            qrh = qr_ref[:, h * DR:(h + 1) * DR]
            s = _dot_nt(qh, kh) + _dot_nt(qrh, kr_all)
            e = jnp.exp(s)
            recip = 1.0 / jnp.sum(e, axis=1, keepdims=True)
            vh = v_ref[:, h * DH:(h + 1) * DH].astype(jnp.float32)
            o_ref[:, h * DH:(h + 1) * DH] = (_dot(e, vh) * recip).astype(bf16)
            r = pltpu.make_async_remote_copy(
                src_ref=o_ref.at[:, pl.ds(h * DH, DH)],
                dst_ref=o_rx_ref.at[:, pl.ds(h * DH, DH)],
                send_sem=send_sems.at[h], recv_sem=recv_sems.at[h],
                device_id=nbr, device_id_type=pl.DeviceIdType.MESH)
            r.start()
            rdmas.append(r)

        cp_wo.wait()
        for side in (0, 1):
            @pl.when(my_x == side)
            def _(side=side):
                lo, hi = side * DL, (side + 1) * DL
                acc_ref[...] = _dot(o_ref[...].astype(jnp.float32),
                                    wo_v_ref[lo:hi, :])

        for r in rdmas:
            r.wait()

        for side in (0, 1):
            @pl.when(my_x == side)
            def _(side=side):
                lo, hi = (1 - side) * DL, (2 - side) * DL
                out_ref[0] = (acc_ref[...]
                              + _dot(o_rx_ref[...].astype(jnp.float32),
                                     wo_v_ref[lo:hi, :])).astype(bf16)

    out = pl.pallas_call(
        body_cd,
        out_shape=jax.ShapeDtypeStruct((1, S, D), bf16),
        in_specs=[_VMEM] * 5
        + [pl.BlockSpec(memory_space=pltpu.MemorySpace.HBM)],
        out_specs=_VMEM,
        scratch_shapes=[
            pltpu.VMEM((S, DL), bf16),
            pltpu.VMEM((S, DL), bf16),
            pltpu.VMEM((D, D), jnp.float32),
            pltpu.VMEM((S, D), jnp.float32),
            pltpu.SemaphoreType.DMA,
            pltpu.SemaphoreType.DMA((HL,)),
            pltpu.SemaphoreType.DMA((HL,)),
        ],
        compiler_params=pltpu.CompilerParams(
            collective_id=1, vmem_limit_bytes=62 * 1024 * 1024),
    )(q, qr, kr, k, v, Wo)

    return out
